# baseline (speedup 1.0000x reference)
import jax
import jax.numpy as jnp
from jax.experimental import pallas as pl
from jax.experimental.pallas import tpu as pltpu

_N = 256
_S = 4


def _fused_kernel(locs_ref, shot_ref, Wg_ref, vs_ref, Ad_ref, bg_ref,
                  Wt0_ref, Wt1_ref, Wt2_ref, bt_ref,
                  W1_ref, b1_ref, W2_ref, b2_ref, Wr_ref, br_ref,
                  shot_out_ref, locs_out_ref, rlocs_ref, rshot_ref):
    f32 = jnp.float32
    row = jax.lax.broadcasted_iota(jnp.int32, (_N, _N), 0)
    col = jax.lax.broadcasted_iota(jnp.int32, (_N, _N), 1)
    diag = row == col
    ridx = jax.lax.broadcasted_iota(jnp.int32, (_N, 16), 0)

    for i in range(_S):
        x = locs_ref[i]
        h = jnp.dot(x, Wg_ref[...], preferred_element_type=f32)

        e_d = jnp.dot(x, Ad_ref[...], preferred_element_type=f32)
        a_s_row = jax.lax.dot_general(
            vs_ref[...], x, (((1,), (1,)), ((), ())),
            preferred_element_type=f32)
        e = e_d + a_s_row
        e = jnp.maximum(e, 0.2 * e)
        e = jnp.where(diag, f32(-1e30), e)
        m = jnp.max(e, axis=1, keepdims=True)
        p = jnp.exp(e - m)
        alpha = p * (1.0 / jnp.sum(p, axis=1, keepdims=True))
        gat = jnp.dot(alpha, h, preferred_element_type=f32)
        gat = jnp.maximum(gat + bg_ref[...], 0.0)

        s0 = shot_ref[i]
        s1 = jnp.where(ridx >= 1, pltpu.roll(s0, 1, 0), 0.0)
        s2 = jnp.where(ridx >= 2, pltpu.roll(s0, 2, 0), 0.0)
        y = (jnp.dot(s0, Wt2_ref[...], preferred_element_type=f32)
             + jnp.dot(s1, Wt1_ref[...], preferred_element_type=f32)
             + jnp.dot(s2, Wt0_ref[...], preferred_element_type=f32))
        tcn = jnp.maximum(y + bt_ref[...], 0.0)

        z = (jnp.dot(gat, W1_ref[0:16, :], preferred_element_type=f32)
             + jnp.dot(tcn, W1_ref[16:32, :], preferred_element_type=f32)
             + b1_ref[...])
        z = jnp.maximum(z, 0.01 * z)
        o2 = jnp.dot(z, W2_ref[...], preferred_element_type=f32) + b2_ref[...]
        r = jnp.dot(o2, Wr_ref[...], preferred_element_type=f32) + br_ref[...]

        shot_out_ref[i] = o2[:, 0:1]
        locs_out_ref[i] = o2[:, 1:2]
        rlocs_ref[i] = r[:, 0:2]
        rshot_ref[i] = r[:, 2:18]


def kernel(locs, shot, W_gat, att_src, att_dst, b_gat, W_tcn, b_tcn,
           W_s1, b_s1, W_s2, b_s2, W_l1, b_l1, W_l2, b_l2,
           W_rl, b_rl, W_rs, b_rs):
    B, N, _ = locs.shape
    f32 = jnp.float32

    row = lambda v: v.reshape(1, -1).astype(f32)
    Wt = jnp.transpose(W_tcn, (1, 0, 2))
    W1 = jnp.concatenate([W_s1, W_l1], axis=1)
    b1 = jnp.concatenate([b_s1, b_l1]).reshape(1, 32)
    z16 = jnp.zeros((16, 1), f32)
    W2 = jnp.concatenate([
        jnp.concatenate([W_s2, z16], axis=1),
        jnp.concatenate([z16, W_l2], axis=1),
    ], axis=0)
    b2 = jnp.concatenate([b_s2, b_l2]).reshape(1, 2)
    Wr = jnp.concatenate([
        jnp.concatenate([jnp.zeros((1, 2), f32), W_rs], axis=1),
        jnp.concatenate([W_rl, jnp.zeros((1, 16), f32)], axis=1),
    ], axis=0)
    br = jnp.concatenate([b_rl, b_rs]).reshape(1, 18)
    args = (
        locs, shot, W_gat,
        row(att_src @ W_gat.T),
        W_gat @ (att_dst.astype(f32)[:, None] * jnp.ones((1, N), f32)),
        row(b_gat),
        Wt[:, :, 0], Wt[:, :, 1], Wt[:, :, 2], row(b_tcn),
        W1, b1, W2, b2, Wr, br,
    )

    batch3 = lambda d: pl.BlockSpec((_S, N, d), lambda b: (b, 0, 0))
    full2 = lambda a: pl.BlockSpec(a.shape, lambda b: (0,) * a.ndim)
    in_specs = [batch3(2), batch3(16)] + [full2(a) for a in args[2:]]

    out_shape = (
        jax.ShapeDtypeStruct((B, N, 1), f32),
        jax.ShapeDtypeStruct((B, N, 1), f32),
        jax.ShapeDtypeStruct((B, N, 2), f32),
        jax.ShapeDtypeStruct((B, N, 16), f32),
    )
    out_specs = (batch3(1), batch3(1), batch3(2), batch3(16))

    return pl.pallas_call(
        _fused_kernel,
        grid=(B // _S,),
        in_specs=in_specs,
        out_specs=out_specs,
        out_shape=out_shape,
        compiler_params=pltpu.CompilerParams(
            dimension_semantics=("parallel",),
        ),
    )(*args)


# speedup vs baseline: 1.1337x; 1.1337x over previous
import jax
import jax.numpy as jnp
from jax.experimental import pallas as pl
from jax.experimental.pallas import tpu as pltpu

_N = 256
_S = 4


def _fused_kernel(locs_ref, shot_ref, Wg_ref, vs_ref, Ad_ref, bg_ref,
                  Wt0_ref, Wt1_ref, Wt2_ref, bt_ref,
                  W1_ref, b1_ref, W2_ref, b2_ref,
                  Wrl_ref, brl_ref, Wrs_ref, brs_ref,
                  shot_out_ref, locs_out_ref, rlocs_ref, rshot_ref):
    f32 = jnp.float32
    row = jax.lax.broadcasted_iota(jnp.int32, (_N, _N), 0)
    col = jax.lax.broadcasted_iota(jnp.int32, (_N, _N), 1)
    ridx = jax.lax.broadcasted_iota(jnp.int32, (_N, 16), 0)

    Wg = Wg_ref[...]
    for i in range(_S):
        x = locs_ref[i]
        h = x[:, 0:1] * Wg[0:1, :] + x[:, 1:2] * Wg[1:2, :]

        e_d = jnp.dot(h, Ad_ref[...], preferred_element_type=f32)
        a_s_row = jax.lax.dot_general(
            vs_ref[...], h, (((1,), (1,)), ((), ())),
            preferred_element_type=f32)
        e = e_d + a_s_row
        e = jnp.where(e >= 0, e, 0.2 * e)
        e = jnp.where(row == col, f32(-1e30), e)
        m = jnp.max(e, axis=1, keepdims=True)
        p = jnp.exp(e - m)
        alpha = p / jnp.sum(p, axis=1, keepdims=True)
        gat = jnp.dot(alpha, h, preferred_element_type=f32)
        gat = jnp.maximum(gat + bg_ref[...], 0.0)

        s0 = shot_ref[i]
        s1 = jnp.where(ridx >= 1, pltpu.roll(s0, 1, 0), 0.0)
        s2 = jnp.where(ridx >= 2, pltpu.roll(s0, 2, 0), 0.0)
        y = (jnp.dot(s0, Wt2_ref[...], preferred_element_type=f32)
             + jnp.dot(s1, Wt1_ref[...], preferred_element_type=f32)
             + jnp.dot(s2, Wt0_ref[...], preferred_element_type=f32))
        tcn = jnp.maximum(y + bt_ref[...], 0.0)

        z = (jnp.dot(gat, W1_ref[0:16, :], preferred_element_type=f32)
             + jnp.dot(tcn, W1_ref[16:32, :], preferred_element_type=f32)
             + b1_ref[...])
        z = jnp.where(z >= 0, z, 0.01 * z)
        o2 = jnp.dot(z, W2_ref[...], preferred_element_type=f32) + b2_ref[...]
        so = o2[:, 0:1]
        lo = o2[:, 1:2]

        shot_out_ref[i] = so
        locs_out_ref[i] = lo
        rlocs_ref[i] = lo * Wrl_ref[...] + brl_ref[...]
        rshot_ref[i] = so * Wrs_ref[...] + brs_ref[...]


def kernel(locs, shot, W_gat, att_src, att_dst, b_gat, W_tcn, b_tcn,
           W_s1, b_s1, W_s2, b_s2, W_l1, b_l1, W_l2, b_l2,
           W_rl, b_rl, W_rs, b_rs):
    B, N, _ = locs.shape
    f32 = jnp.float32

    row = lambda v: v.reshape(1, -1).astype(f32)
    Wt = jnp.transpose(W_tcn, (1, 0, 2))
    W1 = jnp.concatenate([W_s1, W_l1], axis=1)
    b1 = jnp.concatenate([b_s1, b_l1]).reshape(1, 32)
    z16 = jnp.zeros((16, 1), f32)
    W2 = jnp.concatenate([
        jnp.concatenate([W_s2, z16], axis=1),
        jnp.concatenate([z16, W_l2], axis=1),
    ], axis=0)
    b2 = jnp.concatenate([b_s2, b_l2]).reshape(1, 2)
    args = (
        locs, shot, W_gat,
        row(att_src),
        (att_dst.astype(f32)[:, None] * jnp.ones((1, N), f32)),
        row(b_gat),
        Wt[:, :, 0], Wt[:, :, 1], Wt[:, :, 2], row(b_tcn),
        W1, b1, W2, b2,
        W_rl, row(b_rl), W_rs, row(b_rs),
    )

    batch3 = lambda d: pl.BlockSpec((_S, N, d), lambda b: (b, 0, 0))
    full2 = lambda a: pl.BlockSpec(a.shape, lambda b: (0,) * a.ndim)
    in_specs = [batch3(2), batch3(16)] + [full2(a) for a in args[2:]]

    out_shape = (
        jax.ShapeDtypeStruct((B, N, 1), f32),
        jax.ShapeDtypeStruct((B, N, 1), f32),
        jax.ShapeDtypeStruct((B, N, 2), f32),
        jax.ShapeDtypeStruct((B, N, 16), f32),
    )
    out_specs = (batch3(1), batch3(1), batch3(2), batch3(16))

    return pl.pallas_call(
        _fused_kernel,
        grid=(B // _S,),
        in_specs=in_specs,
        out_specs=out_specs,
        out_shape=out_shape,
        compiler_params=pltpu.CompilerParams(
            dimension_semantics=("parallel",),
        ),
    )(*args)
